# baseline (speedup 1.0000x reference)
import functools

import jax
import jax.numpy as jnp
from jax import lax
from jax.experimental import pallas as pl
from jax.experimental.pallas import tpu as pltpu
from jax.experimental.pallas import tpu_sc as plsc

_NW = 32
_CW = 128
_BB = 128


def _gelu(x):
    return 0.5 * x * (1.0 + lax.erf(x * 0.7071067811865476))


def _sc_gather(table, ids3):
    NW, CH, CW = ids3.shape
    d = table.shape[1]
    n = NW * CH * CW
    mesh = plsc.VectorSubcoreMesh(core_axis_name="c", subcore_axis_name="s")

    @functools.partial(
        pl.kernel,
        mesh=mesh,
        out_type=jax.ShapeDtypeStruct((n, d), jnp.float32),
        scratch_types=[
            pltpu.VMEM((CH, CW), jnp.int32),
            pltpu.VMEM((CW, d), jnp.float32),
            pltpu.VMEM((CW, d), jnp.float32),
            pltpu.SemaphoreType.DMA,
            pltpu.SemaphoreType.DMA,
        ],
    )
    def gather_kernel(table_hbm, idx_hbm, out_hbm, idx_v, buf0, buf1, sem0, sem1):
        wid = lax.axis_index("s") * 2 + lax.axis_index("c")
        pltpu.sync_copy(idx_hbm.at[wid], idx_v)
        bufs = (buf0, buf1)
        sems = (sem0, sem1)
        cps = [
            pltpu.async_copy(table_hbm.at[idx_v.at[0]], buf0, sem0),
            pltpu.async_copy(table_hbm.at[idx_v.at[1]], buf1, sem1),
        ]
        for c in range(CH):
            b = c % 2
            cps[b].wait()
            pltpu.sync_copy(bufs[b], out_hbm.at[pl.ds((wid * CH + c) * CW, CW)])
            if c + 2 < CH:
                cps[b] = pltpu.async_copy(table_hbm.at[idx_v.at[c + 2]], bufs[b], sems[b])

    return gather_kernel(table, ids3)


def _tc_body(actor_ref, small_ref, genre_ref, rtab_ref,
             cW1_ref, cb1_ref, cW2_ref, cb2_ref, gW_ref, gb_ref,
             fW1a_ref, fW1b_ref, fW1c_ref, fW1g_ref, fb1_ref, fW2_ref, fb2_ref,
             out_ref):
    f32 = jnp.float32
    bf16 = jnp.bfloat16
    bb, seq, hdim = out_ref.shape
    t = bb * seq
    nc = cW1_ref.shape[0]
    small2 = small_ref[...].reshape(t, small_ref.shape[2])
    career2 = small2[:, :nc]
    rf = small2[:, nc:nc + 1]
    genre2 = genre_ref[...].reshape(t, genre_ref.shape[2])
    c1 = _gelu(jnp.dot(career2, cW1_ref[...], preferred_element_type=f32)
               + cb1_ref[...])
    career_emb = jnp.dot(c1, cW2_ref[...], preferred_element_type=f32) + cb2_ref[...]
    genre_emb = (jnp.dot(genre2, gW_ref[...], preferred_element_type=f32)
                 + gb_ref[...])
    h = jnp.dot(actor_ref[...].astype(bf16), fW1a_ref[...],
                preferred_element_type=f32)
    h = h + jnp.dot(career_emb.astype(bf16), fW1c_ref[...],
                    preferred_element_type=f32)
    h = h + jnp.dot(genre_emb.astype(bf16), fW1g_ref[...],
                    preferred_element_type=f32)
    rT = jnp.dot(rtab_ref[...], fW1b_ref[...], preferred_element_type=f32)
    for k in range(rtab_ref.shape[0]):
        h = h + jnp.where(rf == k, 1.0, 0.0) * rT[k:k + 1, :]
    h = _gelu(h + fb1_ref[...])
    out = jnp.dot(h.astype(bf16), fW2_ref[...], preferred_element_type=f32)
    out = out + fb2_ref[...]
    out_ref[...] = out.reshape(bb, seq, hdim)


def _tc_fused(actor_emb, small, genre, role_table, cW1, cb1, cW2, cb2,
              gW, gb, fW1a, fW1b, fW1c, fW1g, fb1, fW2, fb2):
    bsz, seq, nsmall = small.shape
    d = actor_emb.shape[1]
    hdim = fW2.shape[1]
    bb = _BB

    def full(a):
        return pl.BlockSpec(a.shape, lambda i: tuple(0 for _ in a.shape))

    in_specs = [
        pl.BlockSpec((bb * seq, d), lambda i: (i, 0)),
        pl.BlockSpec((bb, seq, nsmall), lambda i: (i, 0, 0)),
        pl.BlockSpec((bb, seq, genre.shape[2]), lambda i: (i, 0, 0)),
        full(role_table), full(cW1), full(cb1), full(cW2), full(cb2),
        full(gW), full(gb), full(fW1a), full(fW1b), full(fW1c), full(fW1g),
        full(fb1), full(fW2), full(fb2),
    ]
    return pl.pallas_call(
        _tc_body,
        grid=(bsz // bb,),
        in_specs=in_specs,
        out_specs=pl.BlockSpec((bb, seq, hdim), lambda i: (i, 0, 0)),
        out_shape=jax.ShapeDtypeStruct((bsz, seq, hdim), jnp.float32),
    )(actor_emb, small, genre, role_table, cW1, cb1, cW2, cb2,
      gW, gb, fW1a, fW1b, fW1c, fW1g, fb1, fW2, fb2)


def kernel(actor_ids, role_types, career_features, genre_distribution,
           actor_table, role_table, cW1, cb1, cW2, cb2, gW, gb,
           fW1, fb1, fW2, fb2):
    bsz, seq = actor_ids.shape
    n = bsz * seq
    d = actor_table.shape[1]
    q = cW1.shape[1]
    ids3 = actor_ids.astype(jnp.int32).reshape(_NW, n // (_NW * _CW), _CW)
    actor_emb = _sc_gather(actor_table, ids3)
    fW1a = fW1[:d]
    fW1b = fW1[d:d + q]
    fW1c = fW1[d + q:d + 2 * q]
    fW1g = fW1[d + 2 * q:]
    small = jnp.concatenate(
        [career_features, role_types[..., None].astype(jnp.float32)], axis=-1)
    bf16 = jnp.bfloat16
    out = _tc_fused(
        actor_emb, small, genre_distribution,
        role_table, cW1, cb1.reshape(1, -1), cW2, cb2.reshape(1, -1),
        gW, gb.reshape(1, -1),
        fW1a.astype(bf16), fW1b, fW1c.astype(bf16), fW1g.astype(bf16),
        fb1.reshape(1, -1), fW2.astype(bf16), fb2.reshape(1, -1))
    return out


# speedup vs baseline: 1.6110x; 1.6110x over previous
import functools

import jax
import jax.numpy as jnp
from jax import lax
from jax.experimental import pallas as pl
from jax.experimental.pallas import tpu as pltpu
from jax.experimental.pallas import tpu_sc as plsc

_NW = 32
_CW = 128
_BB = 128


def _gelu(x):
    return 0.5 * x * (1.0 + lax.erf(x * 0.7071067811865476))


def _sc_gather(table, ids3):
    NW, CH, CW = ids3.shape
    d = table.shape[1]
    n = NW * CH * CW
    mesh = plsc.VectorSubcoreMesh(core_axis_name="c", subcore_axis_name="s")

    @functools.partial(
        pl.kernel,
        mesh=mesh,
        out_type=jax.ShapeDtypeStruct((n, d), jnp.float32),
        scratch_types=[
            pltpu.VMEM((CH, CW), jnp.int32),
            pltpu.VMEM((CW, d), jnp.float32),
            pltpu.VMEM((CW, d), jnp.float32),
            pltpu.SemaphoreType.DMA,
            pltpu.SemaphoreType.DMA,
        ],
    )
    def gather_kernel(table_hbm, idx_hbm, out_hbm, idx_v, buf0, buf1, sem0, sem1):
        wid = lax.axis_index("s") * 2 + lax.axis_index("c")
        pltpu.sync_copy(idx_hbm.at[wid], idx_v)
        bufs = (buf0, buf1)
        sems = (sem0, sem1)
        cps = [
            pltpu.async_copy(table_hbm.at[idx_v.at[0]], buf0, sem0),
            pltpu.async_copy(table_hbm.at[idx_v.at[1]], buf1, sem1),
        ]
        for c in range(CH):
            b = c % 2
            cps[b].wait()
            pltpu.sync_copy(bufs[b], out_hbm.at[pl.ds((wid * CH + c) * CW, CW)])
            if c + 2 < CH:
                cps[b] = pltpu.async_copy(table_hbm.at[idx_v.at[c + 2]], bufs[b], sems[b])

    return gather_kernel(table, ids3)


def _tc_body(actor_ref, small_ref, rtab_ref,
             Win_ref, bin_ref, cW2_ref, cb2_ref, fW1b_ref, fW1acg_ref,
             fb1_ref, fW2_ref, fb2_ref, out_ref):
    f32 = jnp.float32
    bb, seq, hdim = out_ref.shape
    t = bb * seq
    q = cW2_ref.shape[0]
    ncg = Win_ref.shape[0]
    nr = rtab_ref.shape[0]
    small2 = small_ref[...].reshape(t, small_ref.shape[2])
    cg2 = small2[:, :ncg]
    ri = small2[:, ncg:ncg + 1].astype(jnp.int32)
    onehot = jnp.where(ri == lax.broadcasted_iota(jnp.int32, (1, nr), 1),
                       1.0, 0.0).astype(f32)
    u = jnp.dot(cg2, Win_ref[...], preferred_element_type=f32) + bin_ref[...]
    c1 = _gelu(u[:, :q])
    career_emb = jnp.dot(c1, cW2_ref[...], preferred_element_type=f32) + cb2_ref[...]
    genre_emb = u[:, q:]
    rT = jnp.dot(rtab_ref[...], fW1b_ref[...], preferred_element_type=f32)
    W1 = jnp.concatenate([fW1acg_ref[...], rT], axis=0)
    comb = jnp.concatenate([actor_ref[...], career_emb, genre_emb, onehot],
                           axis=-1)
    h = _gelu(jnp.dot(comb, W1, preferred_element_type=f32) + fb1_ref[...])
    out = jnp.dot(h, fW2_ref[...], preferred_element_type=f32) + fb2_ref[...]
    out_ref[...] = out.reshape(bb, seq, hdim)


def _tc_fused(actor_emb, small, role_table, Win, bin_, cW2, cb2, fW1b, fW1acg,
              fb1, fW2, fb2):
    bsz, seq, nsmall = small.shape
    d = actor_emb.shape[1]
    hdim = fW2.shape[1]
    bb = _BB

    def full(a):
        return pl.BlockSpec(a.shape, lambda i: tuple(0 for _ in a.shape))

    in_specs = [
        pl.BlockSpec((bb * seq, d), lambda i: (i, 0)),
        pl.BlockSpec((bb, seq, nsmall), lambda i: (i, 0, 0)),
        full(role_table), full(Win), full(bin_), full(cW2), full(cb2),
        full(fW1b), full(fW1acg), full(fb1), full(fW2), full(fb2),
    ]
    return pl.pallas_call(
        _tc_body,
        grid=(bsz // bb,),
        in_specs=in_specs,
        out_specs=pl.BlockSpec((bb, seq, hdim), lambda i: (i, 0, 0)),
        out_shape=jax.ShapeDtypeStruct((bsz, seq, hdim), jnp.float32),
    )(actor_emb, small, role_table, Win, bin_, cW2, cb2, fW1b, fW1acg,
      fb1, fW2, fb2)


def kernel(actor_ids, role_types, career_features, genre_distribution,
           actor_table, role_table, cW1, cb1, cW2, cb2, gW, gb,
           fW1, fb1, fW2, fb2):
    bsz, seq = actor_ids.shape
    n = bsz * seq
    d = actor_table.shape[1]
    q = cW1.shape[1]
    ids3 = actor_ids.astype(jnp.int32).reshape(_NW, n // (_NW * _CW), _CW)
    actor_emb = _sc_gather(actor_table, ids3)
    fW1a = fW1[:d]
    fW1b = fW1[d:d + q]
    fW1c = fW1[d + q:d + 2 * q]
    fW1g = fW1[d + 2 * q:]
    small = jnp.concatenate(
        [career_features, genre_distribution,
         role_types[..., None].astype(jnp.float32)], axis=-1)
    nc = cW1.shape[0]
    ng = gW.shape[0]
    Win = jnp.zeros((nc + ng, 2 * q), dtype=jnp.float32)
    Win = Win.at[:nc, :q].set(cW1).at[nc:, q:].set(gW)
    bin_ = jnp.concatenate([cb1, gb]).reshape(1, -1)
    fW1acg = jnp.concatenate([fW1a, fW1c, fW1g], axis=0)
    out = _tc_fused(
        actor_emb, small, role_table, Win, bin_, cW2, cb2.reshape(1, -1),
        fW1b, fW1acg, fb1.reshape(1, -1), fW2, fb2.reshape(1, -1))
    return out


# speedup vs baseline: 1.7863x; 1.1088x over previous
import functools

import jax
import jax.numpy as jnp
from jax import lax
from jax.experimental import pallas as pl
from jax.experimental.pallas import tpu as pltpu
from jax.experimental.pallas import tpu_sc as plsc

_NW = 32
_CW = 128
_BB = 128


def _gelu(x):
    return 0.5 * x * (1.0 + lax.erf(x * 0.7071067811865476))


def _sc_gather(table, ids3):
    NW, CH, CW = ids3.shape
    d = table.shape[1]
    n = NW * CH * CW
    mesh = plsc.VectorSubcoreMesh(core_axis_name="c", subcore_axis_name="s")

    @functools.partial(
        pl.kernel,
        mesh=mesh,
        out_type=jax.ShapeDtypeStruct((n, d), jnp.float32),
        scratch_types=[
            pltpu.VMEM((CH, CW), jnp.int32),
            pltpu.VMEM((CW, d), jnp.float32),
            pltpu.VMEM((CW, d), jnp.float32),
            pltpu.SemaphoreType.DMA,
            pltpu.SemaphoreType.DMA,
        ],
    )
    def gather_kernel(table_hbm, idx_hbm, out_hbm, idx_v, buf0, buf1, sem0, sem1):
        wid = lax.axis_index("s") * 2 + lax.axis_index("c")
        pltpu.sync_copy(idx_hbm.at[wid], idx_v)
        bufs = (buf0, buf1)
        sems = (sem0, sem1)
        cps = [
            pltpu.async_copy(table_hbm.at[idx_v.at[0]], buf0, sem0),
            pltpu.async_copy(table_hbm.at[idx_v.at[1]], buf1, sem1),
        ]
        for c in range(CH):
            b = c % 2
            cps[b].wait()
            pltpu.sync_copy(bufs[b], out_hbm.at[pl.ds((wid * CH + c) * CW, CW)])
            if c + 2 < CH:
                cps[b] = pltpu.async_copy(table_hbm.at[idx_v.at[c + 2]], bufs[b], sems[b])

    return gather_kernel(table, ids3)


def _tc_body(actor_ref, career_ref, genre_ref, role_ref, rtab_ref,
             Win_ref, bin_ref, cW2_ref, cb2_ref, fW1b_ref, fW1acg_ref,
             fb1_ref, fW2_ref, fb2_ref, out_ref):
    f32 = jnp.float32
    bb, seq, hdim = out_ref.shape
    t = bb * seq
    q = cW2_ref.shape[0]
    nr = rtab_ref.shape[0]
    career2 = career_ref[...].reshape(t, career_ref.shape[2])
    genre2 = genre_ref[...].reshape(t, genre_ref.shape[2])
    cg2 = jnp.concatenate([career2, genre2], axis=-1)
    ri3 = lax.broadcast_in_dim(role_ref[...], (bb, seq, nr), (0, 1))
    oh3 = jnp.where(ri3 == lax.broadcasted_iota(jnp.int32, (bb, seq, nr), 2),
                    1.0, 0.0).astype(f32)
    onehot = oh3.reshape(t, nr)
    u = jnp.dot(cg2, Win_ref[...], preferred_element_type=f32) + bin_ref[...]
    c1 = _gelu(u[:, :q])
    career_emb = jnp.dot(c1, cW2_ref[...], preferred_element_type=f32) + cb2_ref[...]
    genre_emb = u[:, q:]
    rT = jnp.dot(rtab_ref[...], fW1b_ref[...], preferred_element_type=f32)
    W1 = jnp.concatenate([fW1acg_ref[...], rT], axis=0)
    comb = jnp.concatenate([actor_ref[...], career_emb, genre_emb, onehot],
                           axis=-1)
    h = _gelu(jnp.dot(comb, W1, preferred_element_type=f32) + fb1_ref[...])
    out = jnp.dot(h, fW2_ref[...], preferred_element_type=f32) + fb2_ref[...]
    out_ref[...] = out.reshape(bb, seq, hdim)


def _tc_fused(actor_emb, career, genre, roles, role_table, Win, bin_, cW2,
              cb2, fW1b, fW1acg, fb1, fW2, fb2):
    bsz, seq, _ = career.shape
    d = actor_emb.shape[1]
    hdim = fW2.shape[1]
    bb = _BB

    def full(a):
        return pl.BlockSpec(a.shape, lambda i: tuple(0 for _ in a.shape))

    in_specs = [
        pl.BlockSpec((bb * seq, d), lambda i: (i, 0)),
        pl.BlockSpec((bb, seq, career.shape[2]), lambda i: (i, 0, 0)),
        pl.BlockSpec((bb, seq, genre.shape[2]), lambda i: (i, 0, 0)),
        pl.BlockSpec((bb, seq), lambda i: (i, 0)),
        full(role_table), full(Win), full(bin_), full(cW2), full(cb2),
        full(fW1b), full(fW1acg), full(fb1), full(fW2), full(fb2),
    ]
    return pl.pallas_call(
        _tc_body,
        grid=(bsz // bb,),
        in_specs=in_specs,
        out_specs=pl.BlockSpec((bb, seq, hdim), lambda i: (i, 0, 0)),
        out_shape=jax.ShapeDtypeStruct((bsz, seq, hdim), jnp.float32),
    )(actor_emb, career, genre, roles, role_table, Win, bin_, cW2, cb2,
      fW1b, fW1acg, fb1, fW2, fb2)


def kernel(actor_ids, role_types, career_features, genre_distribution,
           actor_table, role_table, cW1, cb1, cW2, cb2, gW, gb,
           fW1, fb1, fW2, fb2):
    bsz, seq = actor_ids.shape
    n = bsz * seq
    d = actor_table.shape[1]
    q = cW1.shape[1]
    ids3 = actor_ids.astype(jnp.int32).reshape(_NW, n // (_NW * _CW), _CW)
    actor_emb = _sc_gather(actor_table, ids3)
    fW1a = fW1[:d]
    fW1b = fW1[d:d + q]
    fW1c = fW1[d + q:d + 2 * q]
    fW1g = fW1[d + 2 * q:]
    nc = cW1.shape[0]
    ng = gW.shape[0]
    Win = jnp.zeros((nc + ng, 2 * q), dtype=jnp.float32)
    Win = Win.at[:nc, :q].set(cW1).at[nc:, q:].set(gW)
    bin_ = jnp.concatenate([cb1, gb]).reshape(1, -1)
    fW1acg = jnp.concatenate([fW1a, fW1c, fW1g], axis=0)
    out = _tc_fused(
        actor_emb, career_features, genre_distribution,
        role_types.astype(jnp.int32), role_table, Win, bin_, cW2,
        cb2.reshape(1, -1), fW1b, fW1acg, fb1.reshape(1, -1), fW2,
        fb2.reshape(1, -1))
    return out


# speedup vs baseline: 1.8097x; 1.0131x over previous
import functools

import jax
import jax.numpy as jnp
from jax import lax
from jax.experimental import pallas as pl
from jax.experimental.pallas import tpu as pltpu
from jax.experimental.pallas import tpu_sc as plsc

_NW = 32
_CW = 128
_BB = 256


def _gelu(x):
    return 0.5 * x * (1.0 + lax.erf(x * 0.7071067811865476))


def _sc_gather(table, ids3):
    NW, CH, CW = ids3.shape
    d = table.shape[1]
    n = NW * CH * CW
    mesh = plsc.VectorSubcoreMesh(core_axis_name="c", subcore_axis_name="s")

    @functools.partial(
        pl.kernel,
        mesh=mesh,
        out_type=jax.ShapeDtypeStruct((n, d), jnp.float32),
        scratch_types=[
            pltpu.VMEM((CH, CW), jnp.int32),
            pltpu.VMEM((CW, d), jnp.float32),
            pltpu.VMEM((CW, d), jnp.float32),
            pltpu.SemaphoreType.DMA,
            pltpu.SemaphoreType.DMA,
        ],
    )
    def gather_kernel(table_hbm, idx_hbm, out_hbm, idx_v, buf0, buf1, sem0, sem1):
        wid = lax.axis_index("s") * 2 + lax.axis_index("c")
        pltpu.sync_copy(idx_hbm.at[wid], idx_v)
        bufs = (buf0, buf1)
        sems = (sem0, sem1)
        cps = [
            pltpu.async_copy(table_hbm.at[idx_v.at[0]], buf0, sem0),
            pltpu.async_copy(table_hbm.at[idx_v.at[1]], buf1, sem1),
        ]
        for c in range(CH):
            b = c % 2
            cps[b].wait()
            pltpu.sync_copy(bufs[b], out_hbm.at[pl.ds((wid * CH + c) * CW, CW)])
            if c + 2 < CH:
                cps[b] = pltpu.async_copy(table_hbm.at[idx_v.at[c + 2]], bufs[b], sems[b])

    return gather_kernel(table, ids3)


def _tc_body(actor_ref, career_ref, genre_ref, role_ref, rtab_ref,
             Win_ref, bin_ref, cW2_ref, cb2_ref, fW1b_ref, fW1acg_ref,
             fb1_ref, fW2_ref, fb2_ref, out_ref):
    f32 = jnp.float32
    bb, seq, hdim = out_ref.shape
    t = bb * seq
    q = cW2_ref.shape[0]
    nr = rtab_ref.shape[0]
    career2 = career_ref[...].reshape(t, career_ref.shape[2])
    genre2 = genre_ref[...].reshape(t, genre_ref.shape[2])
    cg2 = jnp.concatenate([career2, genre2], axis=-1)
    ri3 = lax.broadcast_in_dim(role_ref[...], (bb, seq, nr), (0, 1))
    oh3 = jnp.where(ri3 == lax.broadcasted_iota(jnp.int32, (bb, seq, nr), 2),
                    1.0, 0.0).astype(f32)
    onehot = oh3.reshape(t, nr)
    u = jnp.dot(cg2, Win_ref[...], preferred_element_type=f32) + bin_ref[...]
    c1 = _gelu(u[:, :q])
    career_emb = jnp.dot(c1, cW2_ref[...], preferred_element_type=f32) + cb2_ref[...]
    genre_emb = u[:, q:]
    rT = jnp.dot(rtab_ref[...], fW1b_ref[...], preferred_element_type=f32)
    W1 = jnp.concatenate([fW1acg_ref[...], rT], axis=0)
    comb = jnp.concatenate([actor_ref[...], career_emb, genre_emb, onehot],
                           axis=-1)
    h = _gelu(jnp.dot(comb, W1, preferred_element_type=f32) + fb1_ref[...])
    out = jnp.dot(h, fW2_ref[...], preferred_element_type=f32) + fb2_ref[...]
    out_ref[...] = out.reshape(bb, seq, hdim)


def _tc_fused(actor_emb, career, genre, roles, role_table, Win, bin_, cW2,
              cb2, fW1b, fW1acg, fb1, fW2, fb2):
    bsz, seq, _ = career.shape
    d = actor_emb.shape[1]
    hdim = fW2.shape[1]
    bb = _BB

    def full(a):
        return pl.BlockSpec(a.shape, lambda i: tuple(0 for _ in a.shape))

    in_specs = [
        pl.BlockSpec((bb * seq, d), lambda i: (i, 0)),
        pl.BlockSpec((bb, seq, career.shape[2]), lambda i: (i, 0, 0)),
        pl.BlockSpec((bb, seq, genre.shape[2]), lambda i: (i, 0, 0)),
        pl.BlockSpec((bb, seq), lambda i: (i, 0)),
        full(role_table), full(Win), full(bin_), full(cW2), full(cb2),
        full(fW1b), full(fW1acg), full(fb1), full(fW2), full(fb2),
    ]
    return pl.pallas_call(
        _tc_body,
        grid=(bsz // bb,),
        in_specs=in_specs,
        out_specs=pl.BlockSpec((bb, seq, hdim), lambda i: (i, 0, 0)),
        out_shape=jax.ShapeDtypeStruct((bsz, seq, hdim), jnp.float32),
    )(actor_emb, career, genre, roles, role_table, Win, bin_, cW2, cb2,
      fW1b, fW1acg, fb1, fW2, fb2)


def kernel(actor_ids, role_types, career_features, genre_distribution,
           actor_table, role_table, cW1, cb1, cW2, cb2, gW, gb,
           fW1, fb1, fW2, fb2):
    bsz, seq = actor_ids.shape
    n = bsz * seq
    d = actor_table.shape[1]
    q = cW1.shape[1]
    ids3 = actor_ids.astype(jnp.int32).reshape(_NW, n // (_NW * _CW), _CW)
    actor_emb = _sc_gather(actor_table, ids3)
    fW1a = fW1[:d]
    fW1b = fW1[d:d + q]
    fW1c = fW1[d + q:d + 2 * q]
    fW1g = fW1[d + 2 * q:]
    nc = cW1.shape[0]
    ng = gW.shape[0]
    Win = jnp.zeros((nc + ng, 2 * q), dtype=jnp.float32)
    Win = Win.at[:nc, :q].set(cW1).at[nc:, q:].set(gW)
    bin_ = jnp.concatenate([cb1, gb]).reshape(1, -1)
    fW1acg = jnp.concatenate([fW1a, fW1c, fW1g], axis=0)
    out = _tc_fused(
        actor_emb, career_features, genre_distribution,
        role_types.astype(jnp.int32), role_table, Win, bin_, cW2,
        cb2.reshape(1, -1), fW1b, fW1acg, fb1.reshape(1, -1), fW2,
        fb2.reshape(1, -1))
    return out


# speedup vs baseline: 1.8159x; 1.0034x over previous
import functools

import jax
import jax.numpy as jnp
from jax import lax
from jax.experimental import pallas as pl
from jax.experimental.pallas import tpu as pltpu
from jax.experimental.pallas import tpu_sc as plsc

_NW = 32
_CW = 128
_BB = 256


def _gelu(x):
    return 0.5 * x * (1.0 + lax.erf(x * 0.7071067811865476))


def _sc_gather(table, ids3):
    NW, CH, CW = ids3.shape
    d = table.shape[1]
    n = NW * CH * CW
    mesh = plsc.VectorSubcoreMesh(core_axis_name="c", subcore_axis_name="s")

    NB = 4

    @functools.partial(
        pl.kernel,
        mesh=mesh,
        out_type=jax.ShapeDtypeStruct((n, d), jnp.float32),
        scratch_types=(
            [pltpu.VMEM((CH, CW), jnp.int32)]
            + [pltpu.VMEM((CW, d), jnp.float32) for _ in range(NB)]
            + [pltpu.SemaphoreType.DMA for _ in range(2 * NB)]
        ),
    )
    def gather_kernel(table_hbm, idx_hbm, out_hbm, idx_v, *rest):
        bufs = rest[:NB]
        gsem = rest[NB:2 * NB]
        ssem = rest[2 * NB:]
        wid = lax.axis_index("s") * 2 + lax.axis_index("c")
        pltpu.sync_copy(idx_hbm.at[wid], idx_v)
        g = [pltpu.async_copy(table_hbm.at[idx_v.at[c]], bufs[c], gsem[c])
             for c in range(min(NB, CH))]
        st = [None] * NB
        for c in range(CH):
            b = c % NB
            g[b].wait()
            st[b] = pltpu.async_copy(
                bufs[b], out_hbm.at[pl.ds((wid * CH + c) * CW, CW)], ssem[b])
            if c + NB < CH:
                st[b].wait()
                g[b] = pltpu.async_copy(
                    table_hbm.at[idx_v.at[c + NB]], bufs[b], gsem[b])
        for c in range(max(0, CH - NB), CH):
            st[c % NB].wait()

    return gather_kernel(table, ids3)


def _tc_body(actor_ref, career_ref, genre_ref, role_ref, rtab_ref,
             Win_ref, bin_ref, cW2_ref, cb2_ref, fW1b_ref, fW1acg_ref,
             fb1_ref, fW2_ref, fb2_ref, out_ref):
    f32 = jnp.float32
    bb, seq, hdim = out_ref.shape
    t = bb * seq
    q = cW2_ref.shape[0]
    nr = rtab_ref.shape[0]
    career2 = career_ref[...].reshape(t, career_ref.shape[2])
    genre2 = genre_ref[...].reshape(t, genre_ref.shape[2])
    cg2 = jnp.concatenate([career2, genre2], axis=-1)
    ri3 = lax.broadcast_in_dim(role_ref[...], (bb, seq, nr), (0, 1))
    oh3 = jnp.where(ri3 == lax.broadcasted_iota(jnp.int32, (bb, seq, nr), 2),
                    1.0, 0.0).astype(f32)
    onehot = oh3.reshape(t, nr)
    u = jnp.dot(cg2, Win_ref[...], preferred_element_type=f32) + bin_ref[...]
    c1 = _gelu(u[:, :q])
    career_emb = jnp.dot(c1, cW2_ref[...], preferred_element_type=f32) + cb2_ref[...]
    genre_emb = u[:, q:]
    rT = jnp.dot(rtab_ref[...], fW1b_ref[...], preferred_element_type=f32)
    W1 = jnp.concatenate([fW1acg_ref[...], rT], axis=0)
    comb = jnp.concatenate([actor_ref[...], career_emb, genre_emb, onehot],
                           axis=-1)
    h = _gelu(jnp.dot(comb, W1, preferred_element_type=f32) + fb1_ref[...])
    out = jnp.dot(h, fW2_ref[...], preferred_element_type=f32) + fb2_ref[...]
    out_ref[...] = out.reshape(bb, seq, hdim)


def _tc_fused(actor_emb, career, genre, roles, role_table, Win, bin_, cW2,
              cb2, fW1b, fW1acg, fb1, fW2, fb2):
    bsz, seq, _ = career.shape
    d = actor_emb.shape[1]
    hdim = fW2.shape[1]
    bb = _BB

    def full(a):
        return pl.BlockSpec(a.shape, lambda i: tuple(0 for _ in a.shape))

    in_specs = [
        pl.BlockSpec((bb * seq, d), lambda i: (i, 0)),
        pl.BlockSpec((bb, seq, career.shape[2]), lambda i: (i, 0, 0)),
        pl.BlockSpec((bb, seq, genre.shape[2]), lambda i: (i, 0, 0)),
        pl.BlockSpec((bb, seq), lambda i: (i, 0)),
        full(role_table), full(Win), full(bin_), full(cW2), full(cb2),
        full(fW1b), full(fW1acg), full(fb1), full(fW2), full(fb2),
    ]
    return pl.pallas_call(
        _tc_body,
        grid=(bsz // bb,),
        in_specs=in_specs,
        out_specs=pl.BlockSpec((bb, seq, hdim), lambda i: (i, 0, 0)),
        out_shape=jax.ShapeDtypeStruct((bsz, seq, hdim), jnp.float32),
    )(actor_emb, career, genre, roles, role_table, Win, bin_, cW2, cb2,
      fW1b, fW1acg, fb1, fW2, fb2)


def kernel(actor_ids, role_types, career_features, genre_distribution,
           actor_table, role_table, cW1, cb1, cW2, cb2, gW, gb,
           fW1, fb1, fW2, fb2):
    bsz, seq = actor_ids.shape
    n = bsz * seq
    d = actor_table.shape[1]
    q = cW1.shape[1]
    ids3 = actor_ids.astype(jnp.int32).reshape(_NW, n // (_NW * _CW), _CW)
    actor_emb = _sc_gather(actor_table, ids3)
    fW1a = fW1[:d]
    fW1b = fW1[d:d + q]
    fW1c = fW1[d + q:d + 2 * q]
    fW1g = fW1[d + 2 * q:]
    nc = cW1.shape[0]
    ng = gW.shape[0]
    Win = jnp.zeros((nc + ng, 2 * q), dtype=jnp.float32)
    Win = Win.at[:nc, :q].set(cW1).at[nc:, q:].set(gW)
    bin_ = jnp.concatenate([cb1, gb]).reshape(1, -1)
    fW1acg = jnp.concatenate([fW1a, fW1c, fW1g], axis=0)
    out = _tc_fused(
        actor_emb, career_features, genre_distribution,
        role_types.astype(jnp.int32), role_table, Win, bin_, cW2,
        cb2.reshape(1, -1), fW1b, fW1acg, fb1.reshape(1, -1), fW2,
        fb2.reshape(1, -1))
    return out


# speedup vs baseline: 3.7950x; 2.0899x over previous
import functools

import jax
import jax.numpy as jnp
from jax import lax
from jax.experimental import pallas as pl
from jax.experimental.pallas import tpu as pltpu
from jax.experimental.pallas import tpu_sc as plsc

_NW = 32
_CW = 128
_BB = 256


def _gelu(x):
    return 0.5 * x * (1.0 + lax.erf(x * 0.7071067811865476))


def _sc_gather(table, ids3):
    NW, CH, CW = ids3.shape
    d = table.shape[1]
    n = NW * CH * CW
    mesh = plsc.VectorSubcoreMesh(core_axis_name="c", subcore_axis_name="s")

    NB = 4

    @functools.partial(
        pl.kernel,
        mesh=mesh,
        out_type=jax.ShapeDtypeStruct((n, d), jnp.float32),
        scratch_types=(
            [pltpu.VMEM((CH, CW), jnp.int32)]
            + [pltpu.VMEM((CW, d), jnp.float32) for _ in range(NB)]
            + [pltpu.SemaphoreType.DMA for _ in range(2 * NB)]
        ),
    )
    def gather_kernel(table_hbm, idx_hbm, out_hbm, idx_v, *rest):
        bufs = rest[:NB]
        gsem = rest[NB:2 * NB]
        ssem = rest[2 * NB:]
        wid = lax.axis_index("s") * 2 + lax.axis_index("c")
        pltpu.sync_copy(idx_hbm.at[wid], idx_v)
        g = [pltpu.async_copy(table_hbm.at[idx_v.at[c]], bufs[c], gsem[c])
             for c in range(min(NB, CH))]
        st = [None] * NB
        for c in range(CH):
            b = c % NB
            g[b].wait()
            st[b] = pltpu.async_copy(
                bufs[b], out_hbm.at[pl.ds((wid * CH + c) * CW, CW)], ssem[b])
            if c + NB < CH:
                st[b].wait()
                g[b] = pltpu.async_copy(
                    table_hbm.at[idx_v.at[c + NB]], bufs[b], gsem[b])
        for c in range(max(0, CH - NB), CH):
            st[c % NB].wait()

    return gather_kernel(table, ids3)


def _tc_body(actor_ref, career_ref, genre_ref, role_ref, rtab_ref,
             Win_ref, bin_ref, cW2_ref, cb2_ref, fW1b_ref, fW1acg_ref,
             fb1_ref, fW2_ref, fb2_ref, out_ref):
    f32 = jnp.float32
    seq, bb, hdim = out_ref.shape
    t = seq * bb
    q = cW2_ref.shape[0]
    nr = rtab_ref.shape[0]
    career2 = career_ref[...].reshape(t, career_ref.shape[2])
    genre2 = genre_ref[...].reshape(t, genre_ref.shape[2])
    cg2 = jnp.concatenate([career2, genre2], axis=-1)
    ri3 = lax.broadcast_in_dim(role_ref[...], (seq, bb, nr), (0, 1))
    oh3 = jnp.where(ri3 == lax.broadcasted_iota(jnp.int32, (seq, bb, nr), 2),
                    1.0, 0.0).astype(f32)
    onehot = oh3.reshape(t, nr)
    u = jnp.dot(cg2, Win_ref[...], preferred_element_type=f32) + bin_ref[...]
    c1 = _gelu(u[:, :q])
    career_emb = jnp.dot(c1, cW2_ref[...], preferred_element_type=f32) + cb2_ref[...]
    genre_emb = u[:, q:]
    rT = jnp.dot(rtab_ref[...], fW1b_ref[...], preferred_element_type=f32)
    W1 = jnp.concatenate([fW1acg_ref[...], rT], axis=0)
    actor2 = actor_ref[...].reshape(t, actor_ref.shape[2])
    comb = jnp.concatenate([actor2, career_emb, genre_emb, onehot], axis=-1)
    h = _gelu(jnp.dot(comb, W1, preferred_element_type=f32) + fb1_ref[...])
    out = jnp.dot(h, fW2_ref[...], preferred_element_type=f32) + fb2_ref[...]
    out_ref[...] = out.reshape(seq, bb, hdim)


def _tc_fused(actor_emb, career, genre, roles, role_table, Win, bin_, cW2,
              cb2, fW1b, fW1acg, fb1, fW2, fb2):
    seq, bsz, _ = career.shape
    d = actor_emb.shape[2]
    hdim = fW2.shape[1]
    bb = _BB

    def full(a):
        return pl.BlockSpec(a.shape, lambda i: tuple(0 for _ in a.shape))

    in_specs = [
        pl.BlockSpec((seq, bb, d), lambda i: (0, i, 0)),
        pl.BlockSpec((seq, bb, career.shape[2]), lambda i: (0, i, 0)),
        pl.BlockSpec((seq, bb, genre.shape[2]), lambda i: (0, i, 0)),
        pl.BlockSpec((seq, bb), lambda i: (0, i)),
        full(role_table), full(Win), full(bin_), full(cW2), full(cb2),
        full(fW1b), full(fW1acg), full(fb1), full(fW2), full(fb2),
    ]
    return pl.pallas_call(
        _tc_body,
        grid=(bsz // bb,),
        in_specs=in_specs,
        out_specs=pl.BlockSpec((seq, bb, hdim), lambda i: (0, i, 0)),
        out_shape=jax.ShapeDtypeStruct((seq, bsz, hdim), jnp.float32),
    )(actor_emb, career, genre, roles, role_table, Win, bin_, cW2, cb2,
      fW1b, fW1acg, fb1, fW2, fb2)


def kernel(actor_ids, role_types, career_features, genre_distribution,
           actor_table, role_table, cW1, cb1, cW2, cb2, gW, gb,
           fW1, fb1, fW2, fb2):
    bsz, seq = actor_ids.shape
    n = bsz * seq
    d = actor_table.shape[1]
    q = cW1.shape[1]
    ids3 = (actor_ids.transpose(1, 0).astype(jnp.int32)
            .reshape(_NW, n // (_NW * _CW), _CW))
    actor_emb = _sc_gather(actor_table, ids3).reshape(seq, bsz, d)
    fW1a = fW1[:d]
    fW1b = fW1[d:d + q]
    fW1c = fW1[d + q:d + 2 * q]
    fW1g = fW1[d + 2 * q:]
    nc = cW1.shape[0]
    ng = gW.shape[0]
    Win = jnp.zeros((nc + ng, 2 * q), dtype=jnp.float32)
    Win = Win.at[:nc, :q].set(cW1).at[nc:, q:].set(gW)
    bin_ = jnp.concatenate([cb1, gb]).reshape(1, -1)
    fW1acg = jnp.concatenate([fW1a, fW1c, fW1g], axis=0)
    out = _tc_fused(
        actor_emb, career_features.transpose(1, 0, 2),
        genre_distribution.transpose(1, 0, 2),
        role_types.transpose(1, 0).astype(jnp.int32), role_table, Win, bin_,
        cW2, cb2.reshape(1, -1), fW1b, fW1acg, fb1.reshape(1, -1), fW2,
        fb2.reshape(1, -1))
    return out.transpose(1, 0, 2)
